# Initial kernel scaffold; baseline (speedup 1.0000x reference)
import jax
import jax.numpy as jnp
from jax.experimental import pallas as pl
from jax.experimental.pallas import tpu as pltpu

SIDE = 512
NPIX = SIDE * SIDE
B = 16
RES = 0.05
AMP, N_SERSIC, R_SERSIC = 20.0, 1.0, 0.25
B_N = 1.999 * N_SERSIC - 0.327
BIGJ = jnp.int32(1 << 28)


def _main_body(lr_ref, img_ref, kd_ref, dx_ref, dy_ref, out_ref, acc_ref):
    r = pl.program_id(0)
    lr = lr_ref[0]
    img = img_ref[0]
    kd = kd_ref[...]

    masked = jnp.where(kd < BIGJ, lr, -1.0)
    m = jnp.max(masked)
    jbest = jnp.min(jnp.where(masked == m, kd, BIGJ))
    jx = jbest & (SIDE - 1)
    jy = jbest >> 9
    xc = (jx.astype(jnp.float32) - SIDE / 2.0) * RES
    yc = ((SIDE - jy).astype(jnp.float32) - SIDE / 2.0) * RES

    dxc = dx_ref[...] - xc
    dyc = dy_ref[...] - yc
    rad = jnp.sqrt(dxc * dxc + dyc * dyc)
    prof = AMP * jnp.exp(-B_N * (rad * (1.0 / R_SERSIC) - 1.0))

    s1 = jnp.sum(prof)
    s2 = jnp.sum(prof * prof)
    s3 = jnp.sum(prof * img)
    s4 = jnp.sum(img)
    s5 = jnp.sum(img * img)
    mn = jnp.min(prof)
    mx = jnp.max(prof)

    @pl.when(r == 0)
    def _init():
        acc_ref[0] = s1
        acc_ref[1] = s2
        acc_ref[2] = s3
        acc_ref[3] = s4
        acc_ref[4] = s5
        acc_ref[5] = mn
        acc_ref[6] = mx

    @pl.when(r > 0)
    def _acc():
        acc_ref[0] += s1
        acc_ref[1] += s2
        acc_ref[2] += s3
        acc_ref[3] += s4
        acc_ref[4] += s5
        acc_ref[5] = jnp.minimum(acc_ref[5], mn)
        acc_ref[6] = jnp.maximum(acc_ref[6], mx)

    @pl.when(r == B - 1)
    def _final():
        t1, t2, t3 = acc_ref[0], acc_ref[1], acc_ref[2]
        t4, t5 = acc_ref[3], acc_ref[4]
        gmn, gmx = acc_ref[5], acc_ref[6]
        a = 1.0 / (gmx - gmn)
        c = a * gmn
        npix = jnp.float32(B * NPIX)
        total = (a * a * t2 - 2.0 * a * c * t1 + c * c * npix
                 - 2.0 * a * t3 + 2.0 * c * t4 + t5)
        out_ref[0, 0] = total / npix


def _sersic_mse(lr, img, kd, dx, dy):
    return pl.pallas_call(
        _main_body,
        grid=(B,),
        in_specs=[
            pl.BlockSpec((1, SIDE, SIDE), lambda r: (r, 0, 0)),
            pl.BlockSpec((1, SIDE, SIDE), lambda r: (r, 0, 0)),
            pl.BlockSpec((SIDE, SIDE), lambda r: (0, 0)),
            pl.BlockSpec((SIDE, SIDE), lambda r: (0, 0)),
            pl.BlockSpec((SIDE, SIDE), lambda r: (0, 0)),
        ],
        out_specs=pl.BlockSpec((1, 1), lambda r: (0, 0)),
        out_shape=jax.ShapeDtypeStruct((1, 1), jnp.float32),
        scratch_shapes=[pltpu.SMEM((8,), jnp.float32)],
    )(lr, img, kd, dx, dy)


def kernel(image, LR, dest_indices, dest_x, dest_y):
    img = image.reshape(B, SIDE, SIDE)
    lr = LR.reshape(B, SIDE, SIDE)
    iota = jnp.arange(NPIX, dtype=jnp.int32)
    lw = jnp.full((NPIX,), -1, jnp.int32).at[dest_indices].max(iota)
    keep = lw[dest_indices] == iota
    kd = jnp.where(keep, dest_indices, BIGJ).reshape(SIDE, SIDE)
    out = _sersic_mse(lr, img, kd, dest_x.reshape(SIDE, SIDE),
                      dest_y.reshape(SIDE, SIDE))
    return out.reshape(())


# speedup vs baseline: 16.4241x; 16.4241x over previous
import jax
import jax.numpy as jnp
from jax.experimental import pallas as pl
from jax.experimental.pallas import tpu as pltpu

SIDE = 512
NPIX = SIDE * SIDE
B = 16
RES = 0.05
AMP, N_SERSIC, R_SERSIC = 20.0, 1.0, 0.25
B_N = 1.999 * N_SERSIC - 0.327
BIGJ = 1 << 28


def _main_body(lr_ref, img_ref, kd_ref, dx_ref, dy_ref, out_ref, acc_ref):
    r = pl.program_id(0)
    lr = lr_ref[0]
    img = img_ref[0]
    kd = kd_ref[...]

    masked = jnp.where(kd < BIGJ, lr, -1.0)
    m = jnp.max(masked)
    jbest = jnp.min(jnp.where(masked == m, kd, BIGJ))
    jx = jbest & (SIDE - 1)
    jy = jbest >> 9
    xc = (jx.astype(jnp.float32) - SIDE / 2.0) * RES
    yc = ((SIDE - jy).astype(jnp.float32) - SIDE / 2.0) * RES

    dxc = dx_ref[...] - xc
    dyc = dy_ref[...] - yc
    rad = jnp.sqrt(dxc * dxc + dyc * dyc)
    prof = AMP * jnp.exp(-B_N * (rad * (1.0 / R_SERSIC) - 1.0))

    s1 = jnp.sum(prof)
    s2 = jnp.sum(prof * prof)
    s3 = jnp.sum(prof * img)
    s4 = jnp.sum(img)
    s5 = jnp.sum(img * img)
    mn = jnp.min(prof)
    mx = jnp.max(prof)

    @pl.when(r == 0)
    def _init():
        acc_ref[0] = s1
        acc_ref[1] = s2
        acc_ref[2] = s3
        acc_ref[3] = s4
        acc_ref[4] = s5
        acc_ref[5] = mn
        acc_ref[6] = mx

    @pl.when(r > 0)
    def _acc():
        acc_ref[0] += s1
        acc_ref[1] += s2
        acc_ref[2] += s3
        acc_ref[3] += s4
        acc_ref[4] += s5
        acc_ref[5] = jnp.minimum(acc_ref[5], mn)
        acc_ref[6] = jnp.maximum(acc_ref[6], mx)

    @pl.when(r == B - 1)
    def _final():
        t1, t2, t3 = acc_ref[0], acc_ref[1], acc_ref[2]
        t4, t5 = acc_ref[3], acc_ref[4]
        gmn, gmx = acc_ref[5], acc_ref[6]
        a = 1.0 / (gmx - gmn)
        c = a * gmn
        npix = jnp.float32(B * NPIX)
        total = (a * a * t2 - 2.0 * a * c * t1 + c * c * npix
                 - 2.0 * a * t3 + 2.0 * c * t4 + t5)
        out_ref[0, 0] = total / npix


def _sersic_mse(lr, img, kd, dx, dy):
    return pl.pallas_call(
        _main_body,
        grid=(B,),
        in_specs=[
            pl.BlockSpec((1, SIDE, SIDE), lambda r: (r, 0, 0)),
            pl.BlockSpec((1, SIDE, SIDE), lambda r: (r, 0, 0)),
            pl.BlockSpec((SIDE, SIDE), lambda r: (0, 0)),
            pl.BlockSpec((SIDE, SIDE), lambda r: (0, 0)),
            pl.BlockSpec((SIDE, SIDE), lambda r: (0, 0)),
        ],
        out_specs=pl.BlockSpec((1, 1), lambda r: (0, 0),
                               memory_space=pltpu.SMEM),
        out_shape=jax.ShapeDtypeStruct((1, 1), jnp.float32),
        scratch_shapes=[pltpu.SMEM((8,), jnp.float32)],
    )(lr, img, kd, dx, dy)


def kernel(image, LR, dest_indices, dest_x, dest_y):
    img = image.reshape(B, SIDE, SIDE)
    lr = LR.reshape(B, SIDE, SIDE)
    iota = jnp.arange(NPIX, dtype=jnp.int32)
    lw = jnp.full((NPIX,), -1, jnp.int32).at[dest_indices].max(iota)
    keep = lw[dest_indices] == iota
    kd = jnp.where(keep, dest_indices, BIGJ).reshape(SIDE, SIDE)
    out = _sersic_mse(lr, img, kd, dest_x.reshape(SIDE, SIDE),
                      dest_y.reshape(SIDE, SIDE))
    return out.reshape(())


# speedup vs baseline: 163.2251x; 9.9382x over previous
import functools

import jax
import jax.numpy as jnp
from jax import lax
from jax.experimental import pallas as pl
from jax.experimental.pallas import tpu as pltpu
from jax.experimental.pallas import tpu_sc as plsc

SIDE = 512
NPIX = SIDE * SIDE
B = 16
RES = 0.05
AMP, N_SERSIC, R_SERSIC = 20.0, 1.0, 0.25
B_N = 1.999 * N_SERSIC - 0.327
BIGJ = 1 << 28

SC_BLK = NPIX // 16
SC_PROC = 3 * SC_BLK
SC_GCH = NPIX // 32

_sc_mesh = plsc.VectorSubcoreMesh(core_axis_name="c", subcore_axis_name="s")


@functools.partial(
    pl.kernel,
    mesh=_sc_mesh,
    out_type=jax.ShapeDtypeStruct((NPIX,), jnp.int32),
    scratch_types=[
        pltpu.VMEM((SC_PROC,), jnp.int32),
        pltpu.VMEM((SC_BLK,), jnp.int32),
        pltpu.VMEM_SHARED((NPIX,), jnp.int32),
        pltpu.VMEM((SC_GCH,), jnp.int32),
        pltpu.VMEM((SC_GCH,), jnp.int32),
        pltpu.VMEM((SC_GCH,), jnp.int32),
        pltpu.SemaphoreType.DMA,
    ],
    compiler_params=pltpu.CompilerParams(needs_layout_passes=False),
)
def _sc_winner(dest_ref, keep_ref, win_ref, lw_ref, lw_sh, dch_ref, gat_ref,
               out_ref, sem):
    c = lax.axis_index("c")
    s = lax.axis_index("s")
    lanes = lax.iota(jnp.int32, 16)

    j_lo = s * SC_BLK
    start = pl.multiple_of(jnp.clip(j_lo - SC_BLK, 0, NPIX - SC_PROC), 16)
    pltpu.sync_copy(dest_ref.at[pl.ds(start, SC_PROC)], win_ref)

    def _scat(k, carry):
        off = k * 16
        d = win_ref[pl.ds(off, 16)]
        msk = (d >= j_lo) & (d < j_lo + SC_BLK)
        loc = jnp.where(msk, d - j_lo, 0)
        plsc.store_scatter(lw_ref, [loc], start + off + lanes, mask=msk)
        return carry

    lax.fori_loop(0, SC_PROC // 16, _scat, 0)

    pltpu.sync_copy(lw_ref, lw_sh.at[pl.ds(j_lo, SC_BLK)])
    plsc.subcore_barrier()

    gbase = (c * 16 + s) * SC_GCH
    pltpu.sync_copy(dest_ref.at[pl.ds(gbase, SC_GCH)], dch_ref)
    pltpu.async_copy(lw_sh.at[dch_ref], gat_ref, sem).wait()

    def _gath(k, carry):
        off = k * 16
        d = dch_ref[pl.ds(off, 16)]
        g = gat_ref[pl.ds(off, 16)]
        out_ref[pl.ds(off, 16)] = jnp.where(g == gbase + off + lanes, d, BIGJ)
        return carry

    lax.fori_loop(0, SC_GCH // 16, _gath, 0)
    pltpu.sync_copy(out_ref, keep_ref.at[pl.ds(gbase, SC_GCH)])


def _main_body(lr_ref, img_ref, kd_ref, dx_ref, dy_ref, out_ref, acc_ref):
    r = pl.program_id(0)
    lr = lr_ref[0]
    img = img_ref[0]
    kd = kd_ref[...]

    masked = jnp.where(kd < BIGJ, lr, -1.0)
    m = jnp.max(masked)
    jbest = jnp.min(jnp.where(masked == m, kd, BIGJ))
    jx = jbest & (SIDE - 1)
    jy = jbest >> 9
    xc = (jx.astype(jnp.float32) - SIDE / 2.0) * RES
    yc = ((SIDE - jy).astype(jnp.float32) - SIDE / 2.0) * RES

    dxc = dx_ref[...] - xc
    dyc = dy_ref[...] - yc
    rad = jnp.sqrt(dxc * dxc + dyc * dyc)
    prof = AMP * jnp.exp(-B_N * (rad * (1.0 / R_SERSIC) - 1.0))

    s1 = jnp.sum(prof)
    s2 = jnp.sum(prof * prof)
    s3 = jnp.sum(prof * img)
    s4 = jnp.sum(img)
    s5 = jnp.sum(img * img)
    mn = jnp.min(prof)
    mx = jnp.max(prof)

    @pl.when(r == 0)
    def _init():
        acc_ref[0] = s1
        acc_ref[1] = s2
        acc_ref[2] = s3
        acc_ref[3] = s4
        acc_ref[4] = s5
        acc_ref[5] = mn
        acc_ref[6] = mx

    @pl.when(r > 0)
    def _acc():
        acc_ref[0] += s1
        acc_ref[1] += s2
        acc_ref[2] += s3
        acc_ref[3] += s4
        acc_ref[4] += s5
        acc_ref[5] = jnp.minimum(acc_ref[5], mn)
        acc_ref[6] = jnp.maximum(acc_ref[6], mx)

    @pl.when(r == B - 1)
    def _final():
        t1, t2, t3 = acc_ref[0], acc_ref[1], acc_ref[2]
        t4, t5 = acc_ref[3], acc_ref[4]
        gmn, gmx = acc_ref[5], acc_ref[6]
        a = 1.0 / (gmx - gmn)
        c = a * gmn
        npix = jnp.float32(B * NPIX)
        total = (a * a * t2 - 2.0 * a * c * t1 + c * c * npix
                 - 2.0 * a * t3 + 2.0 * c * t4 + t5)
        out_ref[0, 0] = total / npix


def _sersic_mse(lr, img, kd, dx, dy):
    return pl.pallas_call(
        _main_body,
        grid=(B,),
        in_specs=[
            pl.BlockSpec((1, SIDE, SIDE), lambda r: (r, 0, 0)),
            pl.BlockSpec((1, SIDE, SIDE), lambda r: (r, 0, 0)),
            pl.BlockSpec((SIDE, SIDE), lambda r: (0, 0)),
            pl.BlockSpec((SIDE, SIDE), lambda r: (0, 0)),
            pl.BlockSpec((SIDE, SIDE), lambda r: (0, 0)),
        ],
        out_specs=pl.BlockSpec((1, 1), lambda r: (0, 0),
                               memory_space=pltpu.SMEM),
        out_shape=jax.ShapeDtypeStruct((1, 1), jnp.float32),
        scratch_shapes=[pltpu.SMEM((8,), jnp.float32)],
    )(lr, img, kd, dx, dy)


def kernel(image, LR, dest_indices, dest_x, dest_y):
    img = image.reshape(B, SIDE, SIDE)
    lr = LR.reshape(B, SIDE, SIDE)
    kd = _sc_winner(dest_indices).reshape(SIDE, SIDE)
    out = _sersic_mse(lr, img, kd, dest_x.reshape(SIDE, SIDE),
                      dest_y.reshape(SIDE, SIDE))
    return out.reshape(())


# speedup vs baseline: 178.5777x; 1.0941x over previous
import functools
import math

import jax
import jax.numpy as jnp
from jax import lax
from jax.experimental import pallas as pl
from jax.experimental.pallas import tpu as pltpu
from jax.experimental.pallas import tpu_sc as plsc

SIDE = 512
NPIX = SIDE * SIDE
B = 16
RES = 0.05
AMP, N_SERSIC, R_SERSIC = 20.0, 1.0, 0.25
B_N = 1.999 * N_SERSIC - 0.327
BIGJ = 1 << 28

SC_BLK = NPIX // 16
SC_MARGIN = 10304
SC_PROC = SC_BLK + 2 * SC_MARGIN
SC_GCH = NPIX // 32
SC_UNROLL = 8
SC_GUNROLL = 4

_sc_mesh = plsc.VectorSubcoreMesh(core_axis_name="c", subcore_axis_name="s")


@functools.partial(
    pl.kernel,
    mesh=_sc_mesh,
    out_type=jax.ShapeDtypeStruct((NPIX,), jnp.int32),
    scratch_types=[
        pltpu.VMEM((SC_PROC,), jnp.int32),
        pltpu.VMEM((SC_BLK,), jnp.int32),
        pltpu.VMEM_SHARED((NPIX,), jnp.int32),
        pltpu.VMEM((SC_GCH,), jnp.int32),
        pltpu.VMEM((SC_GCH,), jnp.int32),
        pltpu.VMEM((SC_GCH,), jnp.int32),
        pltpu.SemaphoreType.DMA,
    ],
    compiler_params=pltpu.CompilerParams(needs_layout_passes=False),
)
def _sc_winner(dest_ref, keep_ref, win_ref, lw_ref, lw_sh, dch_ref, gat_ref,
               out_ref, sem):
    c = lax.axis_index("c")
    s = lax.axis_index("s")
    lanes = lax.iota(jnp.int32, 16)

    j_lo = s * SC_BLK
    start = pl.multiple_of(
        jnp.clip(j_lo - SC_MARGIN, 0, NPIX - SC_PROC), 16)
    pltpu.sync_copy(dest_ref.at[pl.ds(start, SC_PROC)], win_ref)

    def _scat(k, carry):
        base = k * (16 * SC_UNROLL)
        for u in range(SC_UNROLL):
            off = base + u * 16
            d = win_ref[pl.ds(off, 16)]
            loc = d - j_lo
            msk = plsc.bitcast(loc, jnp.uint32) < jnp.uint32(SC_BLK)
            plsc.store_scatter(lw_ref, [loc], start + off + lanes, mask=msk)
        return carry

    lax.fori_loop(0, SC_PROC // (16 * SC_UNROLL), _scat, 0)

    pltpu.sync_copy(lw_ref, lw_sh.at[pl.ds(j_lo, SC_BLK)])
    plsc.subcore_barrier()

    gbase = (c * 16 + s) * SC_GCH
    pltpu.sync_copy(dest_ref.at[pl.ds(gbase, SC_GCH)], dch_ref)
    pltpu.async_copy(lw_sh.at[dch_ref], gat_ref, sem).wait()

    def _gath(k, carry):
        base = k * (16 * SC_GUNROLL)
        for u in range(SC_GUNROLL):
            off = base + u * 16
            d = dch_ref[pl.ds(off, 16)]
            g = gat_ref[pl.ds(off, 16)]
            out_ref[pl.ds(off, 16)] = jnp.where(
                g == gbase + off + lanes, d, BIGJ)
        return carry

    lax.fori_loop(0, SC_GCH // (16 * SC_GUNROLL), _gath, 0)
    pltpu.sync_copy(out_ref, keep_ref.at[pl.ds(gbase, SC_GCH)])


def _main_body(lr_ref, img_ref, kd_ref, dx_ref, dy_ref, out_ref, acc_ref):
    r = pl.program_id(0)
    lr = lr_ref[0]
    img = img_ref[0]
    kd = kd_ref[...]

    masked = jnp.where(kd < BIGJ, lr, -1.0)
    m = jnp.max(masked)
    jbest = jnp.min(jnp.where(masked == m, kd, BIGJ))
    jx = jbest & (SIDE - 1)
    jy = jbest >> 9
    xc = (jx.astype(jnp.float32) - SIDE / 2.0) * RES
    yc = ((SIDE - jy).astype(jnp.float32) - SIDE / 2.0) * RES

    dxc = dx_ref[...] - xc
    dyc = dy_ref[...] - yc
    rad = jnp.sqrt(dxc * dxc + dyc * dyc)
    k2 = -B_N * (1.0 / R_SERSIC) * math.log2(math.e)
    k1 = math.log2(AMP) + B_N * math.log2(math.e)
    prof = jnp.exp2(k1 + k2 * rad)

    s1 = jnp.sum(prof)
    s2 = jnp.sum(prof * prof)
    s3 = jnp.sum(prof * img)
    s4 = jnp.sum(img)
    s5 = jnp.sum(img * img)
    mn = jnp.min(prof)
    mx = jnp.max(prof)

    @pl.when(r == 0)
    def _init():
        acc_ref[0] = s1
        acc_ref[1] = s2
        acc_ref[2] = s3
        acc_ref[3] = s4
        acc_ref[4] = s5
        acc_ref[5] = mn
        acc_ref[6] = mx

    @pl.when(r > 0)
    def _acc():
        acc_ref[0] += s1
        acc_ref[1] += s2
        acc_ref[2] += s3
        acc_ref[3] += s4
        acc_ref[4] += s5
        acc_ref[5] = jnp.minimum(acc_ref[5], mn)
        acc_ref[6] = jnp.maximum(acc_ref[6], mx)

    @pl.when(r == B - 1)
    def _final():
        t1, t2, t3 = acc_ref[0], acc_ref[1], acc_ref[2]
        t4, t5 = acc_ref[3], acc_ref[4]
        gmn, gmx = acc_ref[5], acc_ref[6]
        a = 1.0 / (gmx - gmn)
        c = a * gmn
        npix = jnp.float32(B * NPIX)
        total = (a * a * t2 - 2.0 * a * c * t1 + c * c * npix
                 - 2.0 * a * t3 + 2.0 * c * t4 + t5)
        out_ref[0, 0] = total / npix


def _sersic_mse(lr, img, kd, dx, dy):
    return pl.pallas_call(
        _main_body,
        grid=(B,),
        in_specs=[
            pl.BlockSpec((1, SIDE, SIDE), lambda r: (r, 0, 0)),
            pl.BlockSpec((1, SIDE, SIDE), lambda r: (r, 0, 0)),
            pl.BlockSpec((SIDE, SIDE), lambda r: (0, 0)),
            pl.BlockSpec((SIDE, SIDE), lambda r: (0, 0)),
            pl.BlockSpec((SIDE, SIDE), lambda r: (0, 0)),
        ],
        out_specs=pl.BlockSpec((1, 1), lambda r: (0, 0),
                               memory_space=pltpu.SMEM),
        out_shape=jax.ShapeDtypeStruct((1, 1), jnp.float32),
        scratch_shapes=[pltpu.SMEM((8,), jnp.float32)],
    )(lr, img, kd, dx, dy)


def kernel(image, LR, dest_indices, dest_x, dest_y):
    img = image.reshape(B, SIDE, SIDE)
    lr = LR.reshape(B, SIDE, SIDE)
    kd = _sc_winner(dest_indices).reshape(SIDE, SIDE)
    out = _sersic_mse(lr, img, kd, dest_x.reshape(SIDE, SIDE),
                      dest_y.reshape(SIDE, SIDE))
    return out.reshape(())


# speedup vs baseline: 213.7655x; 1.1970x over previous
import functools
import math

import jax
import jax.numpy as jnp
from jax import lax
from jax.experimental import pallas as pl
from jax.experimental.pallas import tpu as pltpu
from jax.experimental.pallas import tpu_sc as plsc

SIDE = 512
NPIX = SIDE * SIDE
B = 16
RES = 0.05
AMP, N_SERSIC, R_SERSIC = 20.0, 1.0, 0.25
B_N = 1.999 * N_SERSIC - 0.327
BIGJ = 1 << 28

SC_BLK = NPIX // 16
SC_MARGIN = 10304
SC_PROC = SC_BLK + 2 * SC_MARGIN
SC_GCH = NPIX // 32
SC_UNROLL = 8
SC_GUNROLL = 4

_sc_mesh = plsc.VectorSubcoreMesh(core_axis_name="c", subcore_axis_name="s")


@functools.partial(
    pl.kernel,
    mesh=_sc_mesh,
    out_type=jax.ShapeDtypeStruct((NPIX,), jnp.int32),
    scratch_types=[
        pltpu.VMEM((SC_PROC,), jnp.int32),
        pltpu.VMEM((SC_BLK,), jnp.int32),
        pltpu.VMEM_SHARED((NPIX,), jnp.int32),
        pltpu.VMEM((SC_GCH,), jnp.int32),
        pltpu.VMEM((SC_GCH,), jnp.int32),
        pltpu.VMEM((SC_GCH,), jnp.int32),
        pltpu.SemaphoreType.DMA,
    ],
    compiler_params=pltpu.CompilerParams(needs_layout_passes=False),
)
def _sc_winner(dest_ref, keep_ref, win_ref, lw_ref, lw_sh, dch_ref, gat_ref,
               out_ref, sem):
    c = lax.axis_index("c")
    s = lax.axis_index("s")
    lanes = lax.iota(jnp.int32, 16)

    j_lo = s * SC_BLK
    start = pl.multiple_of(
        jnp.clip(j_lo - SC_MARGIN, 0, NPIX - SC_PROC), 16)
    pltpu.sync_copy(dest_ref.at[pl.ds(start, SC_PROC)], win_ref)

    def _scat(k, carry):
        base = k * (16 * SC_UNROLL)
        ds = [win_ref[pl.ds(base + u * 16, 16)] for u in range(SC_UNROLL)]
        locs = [d - j_lo for d in ds]
        msks = [plsc.bitcast(loc, jnp.uint32) < jnp.uint32(SC_BLK)
                for loc in locs]
        for u in range(SC_UNROLL):
            plsc.store_scatter(lw_ref, [locs[u]],
                               start + base + u * 16 + lanes, mask=msks[u])
        return carry

    lax.fori_loop(0, SC_PROC // (16 * SC_UNROLL), _scat, 0)

    pltpu.sync_copy(lw_ref, lw_sh.at[pl.ds(j_lo, SC_BLK)])
    plsc.subcore_barrier()

    gbase = (c * 16 + s) * SC_GCH
    pltpu.sync_copy(dest_ref.at[pl.ds(gbase, SC_GCH)], dch_ref)
    pltpu.async_copy(lw_sh.at[dch_ref], gat_ref, sem).wait()

    def _gath(k, carry):
        base = k * (16 * SC_GUNROLL)
        ds = [dch_ref[pl.ds(base + u * 16, 16)] for u in range(SC_GUNROLL)]
        gs = [gat_ref[pl.ds(base + u * 16, 16)] for u in range(SC_GUNROLL)]
        for u in range(SC_GUNROLL):
            off = base + u * 16
            out_ref[pl.ds(off, 16)] = jnp.where(
                gs[u] == gbase + off + lanes, ds[u], BIGJ)
        return carry

    lax.fori_loop(0, SC_GCH // (16 * SC_GUNROLL), _gath, 0)
    pltpu.sync_copy(out_ref, keep_ref.at[pl.ds(gbase, SC_GCH)])


def _main_body(lr_ref, img_ref, kd_ref, dx_ref, dy_ref, out_ref, acc_ref):
    r = pl.program_id(0)
    lr = lr_ref[0]
    img = img_ref[0]
    kd = kd_ref[...]

    masked = jnp.where(kd < BIGJ, lr, -1.0)
    m = jnp.max(masked)
    jbest = jnp.min(jnp.where(masked == m, kd, BIGJ))
    jx = jbest & (SIDE - 1)
    jy = jbest >> 9
    xc = (jx.astype(jnp.float32) - SIDE / 2.0) * RES
    yc = ((SIDE - jy).astype(jnp.float32) - SIDE / 2.0) * RES

    dxc = dx_ref[...] - xc
    dyc = dy_ref[...] - yc
    rad = jnp.sqrt(dxc * dxc + dyc * dyc)
    k2 = -B_N * (1.0 / R_SERSIC) * math.log2(math.e)
    k1 = math.log2(AMP) + B_N * math.log2(math.e)
    prof = jnp.exp2(k1 + k2 * rad)

    s1 = jnp.sum(prof)
    s2 = jnp.sum(prof * prof)
    s3 = jnp.sum(prof * img)
    s4 = jnp.sum(img)
    s5 = jnp.sum(img * img)
    mn = jnp.min(prof)
    mx = jnp.max(prof)

    @pl.when(r == 0)
    def _init():
        acc_ref[0] = s1
        acc_ref[1] = s2
        acc_ref[2] = s3
        acc_ref[3] = s4
        acc_ref[4] = s5
        acc_ref[5] = mn
        acc_ref[6] = mx

    @pl.when(r > 0)
    def _acc():
        acc_ref[0] += s1
        acc_ref[1] += s2
        acc_ref[2] += s3
        acc_ref[3] += s4
        acc_ref[4] += s5
        acc_ref[5] = jnp.minimum(acc_ref[5], mn)
        acc_ref[6] = jnp.maximum(acc_ref[6], mx)

    @pl.when(r == B - 1)
    def _final():
        t1, t2, t3 = acc_ref[0], acc_ref[1], acc_ref[2]
        t4, t5 = acc_ref[3], acc_ref[4]
        gmn, gmx = acc_ref[5], acc_ref[6]
        a = 1.0 / (gmx - gmn)
        c = a * gmn
        npix = jnp.float32(B * NPIX)
        total = (a * a * t2 - 2.0 * a * c * t1 + c * c * npix
                 - 2.0 * a * t3 + 2.0 * c * t4 + t5)
        out_ref[0, 0] = total / npix


def _sersic_mse(lr, img, kd, dx, dy):
    return pl.pallas_call(
        _main_body,
        grid=(B,),
        in_specs=[
            pl.BlockSpec((1, SIDE, SIDE), lambda r: (r, 0, 0)),
            pl.BlockSpec((1, SIDE, SIDE), lambda r: (r, 0, 0)),
            pl.BlockSpec((SIDE, SIDE), lambda r: (0, 0)),
            pl.BlockSpec((SIDE, SIDE), lambda r: (0, 0)),
            pl.BlockSpec((SIDE, SIDE), lambda r: (0, 0)),
        ],
        out_specs=pl.BlockSpec((1, 1), lambda r: (0, 0),
                               memory_space=pltpu.SMEM),
        out_shape=jax.ShapeDtypeStruct((1, 1), jnp.float32),
        scratch_shapes=[pltpu.SMEM((8,), jnp.float32)],
    )(lr, img, kd, dx, dy)


def kernel(image, LR, dest_indices, dest_x, dest_y):
    img = image.reshape(B, SIDE, SIDE)
    lr = LR.reshape(B, SIDE, SIDE)
    kd = _sc_winner(dest_indices).reshape(SIDE, SIDE)
    out = _sersic_mse(lr, img, kd, dest_x.reshape(SIDE, SIDE),
                      dest_y.reshape(SIDE, SIDE))
    return out.reshape(())
